# Initial kernel scaffold; baseline (speedup 1.0000x reference)
import jax
import jax.numpy as jnp
from jax.experimental import pallas as pl


def kernel(u_i, u_j, label, embeddings, context_embedding):
    raise NotImplementedError("write your pallas kernel here")


# speedup vs baseline: 1.0870x; 1.0870x over previous
import functools

import jax
import jax.numpy as jnp
from jax import lax
from jax.experimental import pallas as pl
from jax.experimental.pallas import tpu as pltpu
from jax.experimental.pallas import tpu_sc as plsc

_B = 16384
_D = 128
_NC = 2
_NS = 16
_L = 16
_NW = _NC * _NS
_RPW = _B // _NW
_CHUNK = 128
_NCHUNK = _RPW // _CHUNK
_GROUPS = _CHUNK // _L

_LOG1P_C = (
    2.4139036e-09, 0.99999967, -0.49998876, 0.33316692, -0.24865821,
    0.19337637, -0.14517646, 0.094703796, -0.047133465, 0.015145372,
    -0.0022880604,
)


def _log1p_unit(e):
    acc = jnp.full((_L,), _LOG1P_C[-1], jnp.float32)
    for c in _LOG1P_C[-2::-1]:
        acc = acc * e + jnp.float32(c)
    return acc


@functools.partial(
    pl.kernel,
    out_type=jax.ShapeDtypeStruct((_NC, _L), jnp.float32),
    mesh=plsc.VectorSubcoreMesh(core_axis_name="c", subcore_axis_name="s"),
    scratch_types=[
        pltpu.VMEM((_NCHUNK, _CHUNK), jnp.int32),
        pltpu.VMEM((_NCHUNK, _CHUNK), jnp.int32),
        pltpu.VMEM((_CHUNK, _D), jnp.float32),
        pltpu.VMEM((_CHUNK, _D), jnp.float32),
        pltpu.VMEM((_RPW // _L, _L), jnp.float32),
        pltpu.VMEM((_L, _L), jnp.float32),
        pltpu.VMEM((_L,), jnp.float32),
        pltpu.VMEM((_L,), jnp.float32),
        pltpu.VMEM_SHARED((_NS, _L), jnp.float32),
        pltpu.SemaphoreType.DMA,
        pltpu.SemaphoreType.DMA,
    ],
    compiler_params=pltpu.CompilerParams(needs_layout_passes=False),
)
def _line_loss(ui_hbm, uj_hbm, lbl_hbm, emb_hbm, ctx_hbm, out_hbm,
               idx_i, idx_j, rows_i, rows_j, lbl_v, tsc, acc_v, out_v,
               shared, sem_i, sem_j):
    c = lax.axis_index("c")
    s = lax.axis_index("s")
    wid = c * _NS + s

    pltpu.sync_copy(ui_hbm.at[pl.ds(wid * _NCHUNK, _NCHUNK)], idx_i)
    pltpu.sync_copy(uj_hbm.at[pl.ds(wid * _NCHUNK, _NCHUNK)], idx_j)
    pltpu.sync_copy(lbl_hbm.at[wid], lbl_v)
    acc_v[...] = jnp.zeros((_L,), jnp.float32)
    lanes = lax.iota(jnp.int32, _L)

    def chunk_body(k, carry):
        cp_i = pltpu.async_copy(emb_hbm.at[idx_i.at[k]], rows_i, sem_i)
        cp_j = pltpu.async_copy(ctx_hbm.at[idx_j.at[k]], rows_j, sem_j)
        cp_i.wait()
        cp_j.wait()

        def group_body(g, inner):
            for r in range(_L):
                row = g * _L + r
                p = rows_i[row, pl.ds(0, _L)] * rows_j[row, pl.ds(0, _L)]
                for q in range(1, _D // _L):
                    p = p + (rows_i[row, pl.ds(q * _L, _L)]
                             * rows_j[row, pl.ds(q * _L, _L)])
                tsc[r, :] = p
            dots = plsc.load_gather(tsc, [lanes, jnp.zeros((_L,), jnp.int32)])
            for m in range(1, _L):
                dots = dots + plsc.load_gather(
                    tsc, [lanes, jnp.full((_L,), m, jnp.int32)])
            t = dots * lbl_v[k * _GROUPS + g]
            e = jnp.exp(-jnp.abs(t))
            ls = jnp.minimum(t, jnp.float32(0.0)) - _log1p_unit(e)
            acc_v[...] = acc_v[...] + ls
            return inner

        return lax.fori_loop(0, _GROUPS, group_body, carry)

    lax.fori_loop(0, _NCHUNK, chunk_body, 0)

    pltpu.sync_copy(acc_v, shared.at[s])
    plsc.subcore_barrier()

    @pl.when(s == 0)
    def _():
        pltpu.sync_copy(shared, tsc)
        tot = tsc[0, :]
        for i in range(1, _NS):
            tot = tot + tsc[i, :]
        val = jnp.sum(tot) * jnp.float32(-1.0 / _B)
        out_v[...] = jnp.full((_L,), val, jnp.float32)
        pltpu.sync_copy(out_v, out_hbm.at[c])


def kernel(u_i, u_j, label, embeddings, context_embedding):
    ui = u_i.astype(jnp.int32).reshape(_NW * _NCHUNK, _CHUNK)
    uj = u_j.astype(jnp.int32).reshape(_NW * _NCHUNK, _CHUNK)
    lbl = label.astype(jnp.float32).reshape(_NW, _RPW // _L, _L)
    out = _line_loss(ui, uj, lbl, embeddings, context_embedding)
    return out[0, 0] + out[1, 0]
